# Initial kernel scaffold; baseline (speedup 1.0000x reference)
import jax
import jax.numpy as jnp
from jax.experimental import pallas as pl


def kernel(h, edge_index):
    raise NotImplementedError("write your pallas kernel here")


# speedup vs baseline: 3.9444x; 3.9444x over previous
import dataclasses
import functools

import jax
import jax.numpy as jnp
from jax import lax
from jax.experimental import pallas as pl
from jax.experimental.pallas import tpu as pltpu
from jax.experimental.pallas import tpu_sc as plsc

_NC = 2
_NS = 16
_NW = _NC * _NS
_L = 16
_W = 128


def kernel(h, edge_index):
    n_nodes, d = h.shape
    e = edge_index.shape[1]
    assert e % _W == 0 and d % _L == 0
    src = edge_index[0].astype(jnp.int32)
    dst = edge_index[1].astype(jnp.int32)
    n_chunks = e // _W
    n_steps = (n_chunks + _NW - 1) // _NW

    mesh = plsc.VectorSubcoreMesh(core_axis_name="c", subcore_axis_name="s")
    cp = pltpu.CompilerParams()
    if "needs_layout_passes" in pltpu.CompilerParams.__dataclass_fields__:
        cp = dataclasses.replace(cp, needs_layout_passes=False)

    @functools.partial(
        pl.kernel,
        out_type=jax.ShapeDtypeStruct((e,), jnp.float32),
        mesh=mesh,
        compiler_params=cp,
        scratch_types=[
            pltpu.VMEM((_W,), jnp.int32),
            pltpu.VMEM((_W,), jnp.int32),
            pltpu.VMEM((_W, d), jnp.float32),
            pltpu.VMEM((_W, d), jnp.float32),
            pltpu.VMEM((_W * _L,), jnp.float32),
            pltpu.VMEM((_W,), jnp.float32),
            pltpu.SemaphoreType.DMA,
            pltpu.SemaphoreType.DMA,
        ],
    )
    def _scores(h_hbm, src_hbm, dst_hbm, out_hbm, si_v, di_v, u_v, v_v, p_v,
                o_v, sem_u, sem_v):
        wid = lax.axis_index("s") * _NC + lax.axis_index("c")
        row_off = lax.iota(jnp.int32, _L) * _L

        @pl.loop(0, n_steps)
        def _(k):
            c = wid + k * _NW

            @pl.when(c < n_chunks)
            def _():
                base = c * _W
                pltpu.sync_copy(src_hbm.at[pl.ds(base, _W)], si_v)
                pltpu.sync_copy(dst_hbm.at[pl.ds(base, _W)], di_v)
                cp_u = pltpu.async_copy(h_hbm.at[si_v], u_v, sem_u)
                cp_v = pltpu.async_copy(h_hbm.at[di_v], v_v, sem_v)
                cp_u.wait()
                cp_v.wait()

                @pl.loop(0, _W)
                def _(i):
                    acc = u_v[i, pl.ds(0, _L)] * v_v[i, pl.ds(0, _L)]
                    for j in range(1, d // _L):
                        acc = acc + (u_v[i, pl.ds(j * _L, _L)]
                                     * v_v[i, pl.ds(j * _L, _L)])
                    p_v[pl.ds(i * _L, _L)] = acc

                @pl.loop(0, _W // _L)
                def _(g):
                    gbase = g * (_L * _L)
                    acc = plsc.load_gather(p_v, [row_off + gbase])
                    for l in range(1, _L):
                        acc = acc + plsc.load_gather(p_v, [row_off + (gbase + l)])
                    o_v[pl.ds(g * _L, _L)] = acc

                pltpu.sync_copy(o_v, out_hbm.at[pl.ds(base, _W)])

    return _scores(h, src, dst)


# speedup vs baseline: 7.9463x; 2.0146x over previous
import dataclasses
import functools

import jax
import jax.numpy as jnp
from jax import lax
from jax.experimental import pallas as pl
from jax.experimental.pallas import tpu as pltpu
from jax.experimental.pallas import tpu_sc as plsc

_NC = 2
_NS = 16
_NW = _NC * _NS
_L = 16
_W = 128


def kernel(h, edge_index):
    n_nodes, d = h.shape
    e = edge_index.shape[1]
    assert e % _W == 0 and d % _L == 0
    src = edge_index[0].astype(jnp.int32)
    dst = edge_index[1].astype(jnp.int32)
    n_chunks = e // _W
    base_chunks = n_chunks // _NW
    rem = n_chunks % _NW
    maxc = base_chunks + (1 if rem else 0)
    if rem:
        pad = jnp.zeros((_W,), jnp.int32)
        src = jnp.concatenate([src, pad])
        dst = jnp.concatenate([dst, pad])

    mesh = plsc.VectorSubcoreMesh(core_axis_name="c", subcore_axis_name="s")
    cp = pltpu.CompilerParams()
    if "needs_layout_passes" in pltpu.CompilerParams.__dataclass_fields__:
        cp = dataclasses.replace(cp, needs_layout_passes=False)

    @functools.partial(
        pl.kernel,
        out_type=jax.ShapeDtypeStruct((e,), jnp.float32),
        mesh=mesh,
        compiler_params=cp,
        scratch_types=[
            pltpu.VMEM((maxc * _W,), jnp.int32),
            pltpu.VMEM((maxc * _W,), jnp.int32),
            pltpu.VMEM((_W, d), jnp.float32),
            pltpu.VMEM((_W, d), jnp.float32),
            pltpu.VMEM((_W, d), jnp.float32),
            pltpu.VMEM((_W, d), jnp.float32),
            pltpu.VMEM((_W * _L,), jnp.float32),
            pltpu.VMEM((_W,), jnp.float32),
            pltpu.SemaphoreType.DMA,
            pltpu.SemaphoreType.DMA,
        ],
    )
    def _scores(h_hbm, src_hbm, dst_hbm, out_hbm, si_v, di_v, u0, v0, u1, v1,
                p_v, o_v, sem0, sem1):
        wid = lax.axis_index("s") * _NC + lax.axis_index("c")
        nt = base_chunks + jnp.where(wid < rem, 1, 0)
        start_c = wid * base_chunks + jnp.minimum(wid, rem)
        ibase = start_c * _W
        row_off = lax.iota(jnp.int32, _L) * _L

        pltpu.sync_copy(src_hbm.at[pl.ds(ibase, maxc * _W)], si_v)
        pltpu.sync_copy(dst_hbm.at[pl.ds(ibase, maxc * _W)], di_v)

        def issue(slot, ub, vb, sem):
            ioff = slot * _W
            pltpu.async_copy(h_hbm.at[si_v.at[pl.ds(ioff, _W)]], ub, sem)
            pltpu.async_copy(h_hbm.at[di_v.at[pl.ds(ioff, _W)]], vb, sem)

        def drain(slot, ub, vb, sem):
            ioff = slot * _W
            pltpu.make_async_copy(
                h_hbm.at[si_v.at[pl.ds(ioff, _W)]], ub, sem).wait()
            pltpu.make_async_copy(
                h_hbm.at[di_v.at[pl.ds(ioff, _W)]], vb, sem).wait()

        def compute(ub, vb):
            @pl.loop(0, _W)
            def _(i):
                acc = ub[i, pl.ds(0, _L)] * vb[i, pl.ds(0, _L)]
                for j in range(1, d // _L):
                    acc = acc + (ub[i, pl.ds(j * _L, _L)]
                                 * vb[i, pl.ds(j * _L, _L)])
                p_v[pl.ds(i * _L, _L)] = acc

            @pl.loop(0, _W // _L)
            def _(g):
                gbase = g * (_L * _L)
                acc = plsc.load_gather(p_v, [row_off + gbase])
                for l in range(1, _L):
                    acc = acc + plsc.load_gather(p_v, [row_off + (gbase + l)])
                o_v[pl.ds(g * _L, _L)] = acc

        issue(0, u0, v0, sem0)
        issue(1, u1, v1, sem1)

        @pl.loop(0, maxc + (maxc & 1), step=2)
        def _(k):
            for b, (ub, vb, sem) in enumerate(((u0, v0, sem0),
                                               (u1, v1, sem1))):
                slot = k + b

                @pl.when(slot < nt)
                def _():
                    drain(slot, ub, vb, sem)
                    compute(ub, vb)

                    @pl.when(slot + 2 < nt)
                    def _():
                        issue(slot + 2, ub, vb, sem)

                    pltpu.sync_copy(
                        o_v, out_hbm.at[pl.ds(ibase + slot * _W, _W)])

    return _scores(h, src, dst)
